# baseline (speedup 1.0000x reference)
import functools

import jax
import jax.numpy as jnp
from jax import lax
from jax.experimental import pallas as pl
from jax.experimental.pallas import tpu as pltpu
from jax.experimental.pallas import tpu_sc as plsc

_B = 16384
_D = 128
_LANES = 16
_NC = 2
_NS = 16
_NW = _NC * _NS
_BPW = _B // _NW
_CHUNK = 128
_NCH = _BPW // _CHUNK
_NEB = 2


def _body(h_hbm, dom_hbm, emb_hbm, out_hbm,
          idx_v, hbufs, ebufs, sem_i, sem_h, sem_e, sem_o):
    wid = lax.axis_index("s") * _NC + lax.axis_index("c")
    base = wid * _BPW

    idx_cp = pltpu.make_async_copy(dom_hbm.at[pl.ds(base, _BPW)], idx_v, sem_i)
    idx_cp.start()
    for ci in range(_NCH):
        pltpu.make_async_copy(
            h_hbm.at[pl.ds(base + ci * _CHUNK, _CHUNK)],
            hbufs[ci], sem_h.at[ci]).start()
    idx_cp.wait()

    def gather(ci):
        return pltpu.async_copy(
            emb_hbm.at[idx_v.at[pl.ds(ci * _CHUNK, _CHUNK)]],
            ebufs[ci % _NEB], sem_e.at[ci % _NEB])

    gads = {ci: gather(ci) for ci in range(_NEB)}

    wbs = []
    for ci in range(_NCH):
        gads[ci].wait()
        pltpu.make_async_copy(
            h_hbm.at[pl.ds(base + ci * _CHUNK, _CHUNK)],
            hbufs[ci], sem_h.at[ci]).wait()

        eb = ebufs[ci % _NEB]
        hb = hbufs[ci]

        def row(r, c2, eb=eb, hb=hb):
            for c in range(_D // _LANES):
                sl = pl.ds(c * _LANES, _LANES)
                plsc.addupdate(hb.at[r, sl], eb[r, sl])
            return c2

        lax.fori_loop(0, _CHUNK, row, 0)

        if ci + _NEB < _NCH:
            gads[ci + _NEB] = gather(ci + _NEB)

        cp = pltpu.make_async_copy(
            hbufs[ci], out_hbm.at[pl.ds(base + ci * _CHUNK, _CHUNK)],
            sem_o.at[ci])
        cp.start()
        wbs.append(cp)

    for ci in range(_NCH):
        wbs[ci].wait()


@jax.jit
def _domain_token(h, domain, emb):
    mesh = plsc.VectorSubcoreMesh(core_axis_name="c", subcore_axis_name="s")
    return pl.kernel(
        _body,
        out_type=jax.ShapeDtypeStruct((_B, _D), jnp.float32),
        mesh=mesh,
        scratch_types=[
            pltpu.VMEM((_BPW,), jnp.int32),
            [pltpu.VMEM((_CHUNK, _D), jnp.float32) for _ in range(_NCH)],
            [pltpu.VMEM((_CHUNK, _D), jnp.float32) for _ in range(_NEB)],
            pltpu.SemaphoreType.DMA,
            pltpu.SemaphoreType.DMA((_NCH,)),
            pltpu.SemaphoreType.DMA((_NEB,)),
            pltpu.SemaphoreType.DMA((_NCH,)),
        ],
    )(h, domain, emb)


def kernel(h, domain, emb):
    return _domain_token(h, domain.astype(jnp.int32), emb)


# speedup vs baseline: 1.0364x; 1.0364x over previous
import functools

import jax
import jax.numpy as jnp
from jax import lax
from jax.experimental import pallas as pl
from jax.experimental.pallas import tpu as pltpu
from jax.experimental.pallas import tpu_sc as plsc

_B = 16384
_D = 128
_NC = 2
_NS = 16
_NW = _NC * _NS
_BPW = _B // _NW
_CHUNK = 256
_NCH = _BPW // _CHUNK


def _body(h_hbm, dom_hbm, emb_hbm, out_hbm,
          idx_v, hbufs, sem_i, sem_h, sem_e, sem_o):
    wid = lax.axis_index("s") * _NC + lax.axis_index("c")
    base = wid * _BPW

    idx_cp = pltpu.make_async_copy(dom_hbm.at[pl.ds(base, _BPW)], idx_v, sem_i)
    idx_cp.start()
    for ci in range(_NCH):
        pltpu.make_async_copy(
            h_hbm.at[pl.ds(base + ci * _CHUNK, _CHUNK)],
            hbufs[ci], sem_h.at[ci]).start()
    idx_cp.wait()

    gadds = []
    for ci in range(_NCH):
        pltpu.make_async_copy(
            h_hbm.at[pl.ds(base + ci * _CHUNK, _CHUNK)],
            hbufs[ci], sem_h.at[ci]).wait()
        cp = pltpu.async_copy(
            emb_hbm.at[idx_v.at[pl.ds(ci * _CHUNK, _CHUNK)]],
            hbufs[ci], sem_e.at[ci], add=True)
        gadds.append(cp)

    wbs = []
    for ci in range(_NCH):
        gadds[ci].wait()
        cp = pltpu.make_async_copy(
            hbufs[ci], out_hbm.at[pl.ds(base + ci * _CHUNK, _CHUNK)],
            sem_o.at[ci])
        cp.start()
        wbs.append(cp)

    for ci in range(_NCH):
        wbs[ci].wait()


@jax.jit
def _domain_token(h, domain, emb):
    mesh = plsc.VectorSubcoreMesh(core_axis_name="c", subcore_axis_name="s")
    return pl.kernel(
        _body,
        out_type=jax.ShapeDtypeStruct((_B, _D), jnp.float32),
        mesh=mesh,
        scratch_types=[
            pltpu.VMEM((_BPW,), jnp.int32),
            [pltpu.VMEM((_CHUNK, _D), jnp.float32) for _ in range(_NCH)],
            pltpu.SemaphoreType.DMA,
            pltpu.SemaphoreType.DMA((_NCH,)),
            pltpu.SemaphoreType.DMA((_NCH,)),
            pltpu.SemaphoreType.DMA((_NCH,)),
        ],
    )(h, domain, emb)


def kernel(h, domain, emb):
    return _domain_token(h, domain.astype(jnp.int32), emb)


# speedup vs baseline: 1.0551x; 1.0181x over previous
import functools

import jax
import jax.numpy as jnp
from jax import lax
from jax.experimental import pallas as pl
from jax.experimental.pallas import tpu as pltpu
from jax.experimental.pallas import tpu_sc as plsc

_B = 16384
_D = 128
_NC = 2
_NS = 16
_NW = _NC * _NS
_BPW = _B // _NW
_CHUNK = 512
_NCH = _BPW // _CHUNK


def _body(h_hbm, dom_hbm, emb_hbm, out_hbm,
          idx_v, hbufs, sem_i, sem_h, sem_e, sem_o):
    wid = lax.axis_index("s") * _NC + lax.axis_index("c")
    base = wid * _BPW

    idx_cp = pltpu.make_async_copy(dom_hbm.at[pl.ds(base, _BPW)], idx_v, sem_i)
    idx_cp.start()
    for ci in range(_NCH):
        pltpu.make_async_copy(
            h_hbm.at[pl.ds(base + ci * _CHUNK, _CHUNK)],
            hbufs[ci], sem_h.at[ci]).start()
    idx_cp.wait()

    gadds = []
    for ci in range(_NCH):
        pltpu.make_async_copy(
            h_hbm.at[pl.ds(base + ci * _CHUNK, _CHUNK)],
            hbufs[ci], sem_h.at[ci]).wait()
        cp = pltpu.async_copy(
            emb_hbm.at[idx_v.at[pl.ds(ci * _CHUNK, _CHUNK)]],
            hbufs[ci], sem_e.at[ci], add=True)
        gadds.append(cp)

    wbs = []
    for ci in range(_NCH):
        gadds[ci].wait()
        cp = pltpu.make_async_copy(
            hbufs[ci], out_hbm.at[pl.ds(base + ci * _CHUNK, _CHUNK)],
            sem_o.at[ci])
        cp.start()
        wbs.append(cp)

    for ci in range(_NCH):
        wbs[ci].wait()


@jax.jit
def _domain_token(h, domain, emb):
    mesh = plsc.VectorSubcoreMesh(core_axis_name="c", subcore_axis_name="s")
    return pl.kernel(
        _body,
        out_type=jax.ShapeDtypeStruct((_B, _D), jnp.float32),
        mesh=mesh,
        scratch_types=[
            pltpu.VMEM((_BPW,), jnp.int32),
            [pltpu.VMEM((_CHUNK, _D), jnp.float32) for _ in range(_NCH)],
            pltpu.SemaphoreType.DMA,
            pltpu.SemaphoreType.DMA((_NCH,)),
            pltpu.SemaphoreType.DMA((_NCH,)),
            pltpu.SemaphoreType.DMA((_NCH,)),
        ],
    )(h, domain, emb)


def kernel(h, domain, emb):
    return _domain_token(h, domain.astype(jnp.int32), emb)


# speedup vs baseline: 1.0653x; 1.0096x over previous
import functools

import jax
import jax.numpy as jnp
from jax import lax
from jax.experimental import pallas as pl
from jax.experimental.pallas import tpu as pltpu
from jax.experimental.pallas import tpu_sc as plsc

_B = 16384
_D = 128
_NC = 2
_NS = 16
_NW = _NC * _NS
_BPW = _B // _NW


def _body(h_hbm, dom_hbm, emb_hbm, out_hbm, idx_v, hbuf, sem_i, sem_h, sem_e,
          sem_o):
    wid = lax.axis_index("s") * _NC + lax.axis_index("c")
    base = wid * _BPW
    rows = pl.ds(base, _BPW)

    idx_cp = pltpu.make_async_copy(dom_hbm.at[rows], idx_v, sem_i)
    idx_cp.start()
    h_cp = pltpu.make_async_copy(h_hbm.at[rows], hbuf, sem_h)
    h_cp.start()
    idx_cp.wait()
    h_cp.wait()

    pltpu.async_copy(emb_hbm.at[idx_v], hbuf, sem_e, add=True).wait()

    out_cp = pltpu.make_async_copy(hbuf, out_hbm.at[rows], sem_o)
    out_cp.start()
    out_cp.wait()


@jax.jit
def _domain_token(h, domain, emb):
    mesh = plsc.VectorSubcoreMesh(core_axis_name="c", subcore_axis_name="s")
    return pl.kernel(
        _body,
        out_type=jax.ShapeDtypeStruct((_B, _D), jnp.float32),
        mesh=mesh,
        scratch_types=[
            pltpu.VMEM((_BPW,), jnp.int32),
            pltpu.VMEM((_BPW, _D), jnp.float32),
            pltpu.SemaphoreType.DMA,
            pltpu.SemaphoreType.DMA,
            pltpu.SemaphoreType.DMA,
            pltpu.SemaphoreType.DMA,
        ],
    )(h, domain, emb)


def kernel(h, domain, emb):
    return _domain_token(h, domain.astype(jnp.int32), emb)
